# baseline (speedup 1.0000x reference)
import functools

import jax
import jax.numpy as jnp
from jax import lax
from jax.experimental import pallas as pl
from jax.experimental.pallas import tpu as pltpu
from jax.experimental.pallas import tpu_sc as plsc

_N = 10000
_E = 320000
_NPAD = 10240
_NC = 2
_NS = 16
_KC = 128
_CPT = 80
_EPAD = _NC * _NS * _CPT * _KC
_RPT = _NPAD // _NS
_ROWBLK = 1024
_GRID = _NPAD // _ROWBLK

_mesh = plsc.VectorSubcoreMesh(core_axis_name="c", subcore_axis_name="s")


def _make_deg_kernel():
    @functools.partial(
        pl.kernel,
        out_type=jax.ShapeDtypeStruct((_NC * _NPAD,), jnp.float32),
        mesh=_mesh,
        scratch_types=[
            pltpu.VMEM((_CPT, _KC), jnp.int32),
            pltpu.VMEM((_NPAD,), jnp.float32),
            pltpu.VMEM((_RPT,), jnp.float32),
            pltpu.VMEM((_RPT,), jnp.float32),
            pltpu.VMEM_SHARED((_NS * _NPAD,), jnp.float32),
        ],
        compiler_params=pltpu.CompilerParams(needs_layout_passes=False),
    )
    def deg_kernel(dst_hbm, zeros_hbm, out_hbm, didx_v, acc_v, tmp_v, red_v, stage_sh):
        cid = lax.axis_index("c")
        sid = lax.axis_index("s")
        wid = sid * _NC + cid
        pltpu.sync_copy(dst_hbm.at[pl.ds(wid * _CPT, _CPT)], didx_v)
        pltpu.sync_copy(zeros_hbm, acc_v)
        ones16 = jnp.ones((16,), jnp.float32)

        def chunk(ci, carry):
            for g in range(_KC // 16):
                idx = didx_v[ci, pl.ds(g * 16, 16)]
                plsc.addupdate_scatter(acc_v, [idx], ones16)
            return carry

        lax.fori_loop(0, _CPT, chunk, 0)
        pltpu.sync_copy(acc_v, stage_sh.at[pl.ds(sid * _NPAD, _NPAD)])
        plsc.subcore_barrier()
        pltpu.sync_copy(stage_sh.at[pl.ds(sid * _RPT, _RPT)], red_v)
        for t in range(1, _NS):
            pltpu.sync_copy(
                stage_sh.at[pl.ds(t * _NPAD + sid * _RPT, _RPT)], tmp_v)
            for q in range(_RPT // 16):
                red_v[pl.ds(q * 16, 16)] = (
                    red_v[pl.ds(q * 16, 16)] + tmp_v[pl.ds(q * 16, 16)])
        pltpu.sync_copy(red_v, out_hbm.at[pl.ds(cid * _NPAD + sid * _RPT, _RPT)])

    return deg_kernel


def _make_scatter_kernel(F):
    @functools.partial(
        pl.kernel,
        out_type=jax.ShapeDtypeStruct((_NC, _NPAD, F), jnp.float32),
        mesh=_mesh,
        scratch_types=[
            pltpu.VMEM((_CPT // 2, _KC), jnp.int32),
            pltpu.VMEM((_CPT // 2, _KC), jnp.int32),
            pltpu.VMEM((2, _KC, F), jnp.float32),
            pltpu.VMEM_SHARED((_NPAD, F), jnp.float32),
            pltpu.SemaphoreType.DMA,
            pltpu.SemaphoreType.DMA,
        ],
    )
    def scat_kernel(vals_hbm, src_hbm, dst_hbm, zeros_hbm, out_hbm,
                    sidx_v, didx_v, rows_v, acc_sh, sem0, sem1):
        cid = lax.axis_index("c")
        sid = lax.axis_index("s")
        wid = sid * _NC + cid
        half = _CPT // 2
        pltpu.sync_copy(zeros_hbm, acc_sh.at[pl.ds(sid * _RPT, _RPT)])
        plsc.subcore_barrier()

        def gather(ci, buf, sem):
            return pltpu.make_async_copy(
                vals_hbm.at[sidx_v.at[ci]], rows_v.at[buf], sem)

        def scatter(ci, buf):
            pltpu.sync_copy(rows_v.at[buf], acc_sh.at[didx_v.at[ci]], add=True)

        for ph in range(2):
            base = wid * _CPT + ph * half
            pltpu.sync_copy(src_hbm.at[pl.ds(base, half)], sidx_v)
            pltpu.sync_copy(dst_hbm.at[pl.ds(base, half)], didx_v)
            gather(0, 0, sem0).start()

            def pipe(j, carry):
                c0 = 2 * j
                gather(c0 + 1, 1, sem1).start()
                gather(c0, 0, sem0).wait()
                scatter(c0, 0)
                gather((c0 + 2) % half, 0, sem0).start()
                gather(c0 + 1, 1, sem1).wait()
                scatter(c0 + 1, 1)
                return carry

            lax.fori_loop(0, half // 2, pipe, 0)
            gather(0, 0, sem0).wait()
        plsc.subcore_barrier()
        pltpu.sync_copy(
            acc_sh.at[pl.ds(sid * _RPT, _RPT)],
            out_hbm.at[cid, pl.ds(sid * _RPT, _RPT)],
        )

    return scat_kernel


_deg_kernel = _make_deg_kernel()
_scat128 = _make_scatter_kernel(128)


def _mm_xw_body(x_ref, w_ref, o_ref):
    o_ref[...] = jnp.dot(x_ref[...], w_ref[...], preferred_element_type=jnp.float32)


def _prep_body(degp_ref, xw_ref, xws_ref, dis_ref, inv_ref):
    deg = degp_ref[0] + degp_ref[1] + 1.0
    dis = lax.rsqrt(deg)
    inv = 1.0 / deg
    xws_ref[...] = xw_ref[...] * dis
    dis_ref[...] = jnp.broadcast_to(dis, (_ROWBLK, 128))
    inv_ref[...] = jnp.broadcast_to(inv, (_ROWBLK, 128))


def _mid_body(s1_ref, xw_ref, dis_ref, inv_ref, wm_ref, bc_ref, m_ref, ms_ref):
    hidden = (dis_ref[...] * (s1_ref[0] + s1_ref[1])
              + xw_ref[...] * inv_ref[...] + bc_ref[...])
    m = jnp.dot(hidden, wm_ref[...], preferred_element_type=jnp.float32)
    m_ref[...] = m
    ms = m * dis_ref[:, :64]
    ms_ref[...] = jnp.concatenate([ms, jnp.zeros_like(ms)], axis=1)


def _fin_body(s2_ref, m_ref, dis_ref, inv_ref, bm_ref, mu_ref, lv_ref):
    out2 = (dis_ref[:, :64] * (s2_ref[0, :, :64] + s2_ref[1, :, :64])
            + m_ref[...] * inv_ref[:, :64] + bm_ref[...])
    mu_ref[...] = out2[:, :32]
    lv_ref[...] = out2[:, 32:]


def _adj_body(mu_ref, mut_ref, o_ref):
    p = jnp.dot(mu_ref[...], mut_ref[...], preferred_element_type=jnp.float32)
    o_ref[...] = jax.nn.sigmoid(p)


def kernel(x, edge_index, W1, b1, W2, b2, Wmu, bmu, Wlv, blv):
    f32 = jnp.float32
    epad = jnp.full((2, _EPAD - _E), _NPAD - 1, jnp.int32)
    ei = jnp.concatenate([jnp.asarray(edge_index, jnp.int32), epad], axis=1)
    src = ei[0].reshape(_EPAD // _KC, _KC)
    dst = ei[1].reshape(_EPAD // _KC, _KC)
    x_pad = jnp.pad(x.astype(f32), ((0, _NPAD - _N), (0, 0)))
    Wc = jnp.concatenate([W1, W2], axis=1).astype(f32)
    bc = jnp.concatenate([b1, b2]).reshape(1, 128).astype(f32)
    Wm = jnp.concatenate([Wmu, Wlv], axis=1).astype(f32)
    bm = jnp.concatenate([bmu, blv]).reshape(1, 64).astype(f32)

    zn = jnp.zeros((_NPAD,), f32)
    z128 = jnp.zeros((_RPT, 128), f32)

    degp = _deg_kernel(dst, zn)
    degp = degp.reshape(_NC, _NPAD, 1)

    xw = pl.pallas_call(
        _mm_xw_body,
        grid=(_GRID,),
        in_specs=[
            pl.BlockSpec((_ROWBLK, 256), lambda i: (i, 0)),
            pl.BlockSpec((256, 128), lambda i: (0, 0)),
        ],
        out_specs=pl.BlockSpec((_ROWBLK, 128), lambda i: (i, 0)),
        out_shape=jax.ShapeDtypeStruct((_NPAD, 128), f32),
    )(x_pad, Wc)

    xws, dis_b, inv_b = pl.pallas_call(
        _prep_body,
        grid=(_GRID,),
        in_specs=[
            pl.BlockSpec((2, _ROWBLK, 1), lambda i: (0, i, 0)),
            pl.BlockSpec((_ROWBLK, 128), lambda i: (i, 0)),
        ],
        out_specs=[
            pl.BlockSpec((_ROWBLK, 128), lambda i: (i, 0)),
            pl.BlockSpec((_ROWBLK, 128), lambda i: (i, 0)),
            pl.BlockSpec((_ROWBLK, 128), lambda i: (i, 0)),
        ],
        out_shape=[
            jax.ShapeDtypeStruct((_NPAD, 128), f32),
            jax.ShapeDtypeStruct((_NPAD, 128), f32),
            jax.ShapeDtypeStruct((_NPAD, 128), f32),
        ],
    )(degp, xw)

    s1p = _scat128(xws, src, dst, z128)

    m, ms = pl.pallas_call(
        _mid_body,
        grid=(_GRID,),
        in_specs=[
            pl.BlockSpec((2, _ROWBLK, 128), lambda i: (0, i, 0)),
            pl.BlockSpec((_ROWBLK, 128), lambda i: (i, 0)),
            pl.BlockSpec((_ROWBLK, 128), lambda i: (i, 0)),
            pl.BlockSpec((_ROWBLK, 128), lambda i: (i, 0)),
            pl.BlockSpec((128, 64), lambda i: (0, 0)),
            pl.BlockSpec((1, 128), lambda i: (0, 0)),
        ],
        out_specs=[
            pl.BlockSpec((_ROWBLK, 64), lambda i: (i, 0)),
            pl.BlockSpec((_ROWBLK, 128), lambda i: (i, 0)),
        ],
        out_shape=[
            jax.ShapeDtypeStruct((_NPAD, 64), f32),
            jax.ShapeDtypeStruct((_NPAD, 128), f32),
        ],
    )(s1p, xw, dis_b, inv_b, Wm, bc)

    s2p = _scat128(ms, src, dst, z128)

    mu_full, lv_full = pl.pallas_call(
        _fin_body,
        grid=(_GRID,),
        in_specs=[
            pl.BlockSpec((2, _ROWBLK, 128), lambda i: (0, i, 0)),
            pl.BlockSpec((_ROWBLK, 64), lambda i: (i, 0)),
            pl.BlockSpec((_ROWBLK, 128), lambda i: (i, 0)),
            pl.BlockSpec((_ROWBLK, 128), lambda i: (i, 0)),
            pl.BlockSpec((1, 64), lambda i: (0, 0)),
        ],
        out_specs=[
            pl.BlockSpec((_ROWBLK, 32), lambda i: (i, 0)),
            pl.BlockSpec((_ROWBLK, 32), lambda i: (i, 0)),
        ],
        out_shape=[
            jax.ShapeDtypeStruct((_NPAD, 32), f32),
            jax.ShapeDtypeStruct((_NPAD, 32), f32),
        ],
    )(s2p, m, dis_b, inv_b, bm)

    mu = mu_full[:_N]
    logvar = lv_full[:_N]
    mut = mu.T

    adj_blk = 200
    adj = pl.pallas_call(
        _adj_body,
        grid=(_N // adj_blk,),
        in_specs=[
            pl.BlockSpec((adj_blk, 32), lambda i: (i, 0)),
            pl.BlockSpec((32, _N), lambda i: (0, 0)),
        ],
        out_specs=pl.BlockSpec((adj_blk, _N), lambda i: (i, 0)),
        out_shape=jax.ShapeDtypeStruct((_N, _N), f32),
    )(mu, mut)

    return (adj, mu, logvar)


# speedup vs baseline: 1.4157x; 1.4157x over previous
import functools

import jax
import jax.numpy as jnp
from jax import lax
from jax.experimental import pallas as pl
from jax.experimental.pallas import tpu as pltpu
from jax.experimental.pallas import tpu_sc as plsc

_N = 10000
_E = 320000
_NPAD = 10240
_NC = 2
_NS = 16
_KC = 128
_CPT = 80
_EPAD = _NC * _NS * _CPT * _KC
_K = 80
_EPW = _E // (_NC * _NS)
_RPT = _NPAD // _NS
_ROWBLK = 1024
_GRID = _NPAD // _ROWBLK

_mesh = plsc.VectorSubcoreMesh(core_axis_name="c", subcore_axis_name="s")


def _make_deg_kernel():
    @functools.partial(
        pl.kernel,
        out_type=jax.ShapeDtypeStruct((_NC * _NPAD,), jnp.float32),
        mesh=_mesh,
        scratch_types=[
            pltpu.VMEM((_CPT, _KC), jnp.int32),
            pltpu.VMEM((_NPAD,), jnp.float32),
            pltpu.VMEM((_RPT,), jnp.float32),
            pltpu.VMEM((_RPT,), jnp.float32),
            pltpu.VMEM_SHARED((_NS * _NPAD,), jnp.float32),
        ],
        compiler_params=pltpu.CompilerParams(needs_layout_passes=False),
    )
    def deg_kernel(dst_hbm, zeros_hbm, out_hbm, didx_v, acc_v, tmp_v, red_v, stage_sh):
        cid = lax.axis_index("c")
        sid = lax.axis_index("s")
        wid = sid * _NC + cid
        pltpu.sync_copy(dst_hbm.at[pl.ds(wid * _CPT, _CPT)], didx_v)
        pltpu.sync_copy(zeros_hbm, acc_v)
        ones16 = jnp.ones((16,), jnp.float32)

        def chunk(ci, carry):
            for g in range(_KC // 16):
                idx = didx_v[ci, pl.ds(g * 16, 16)]
                plsc.addupdate_scatter(acc_v, [idx], ones16)
            return carry

        lax.fori_loop(0, _CPT, chunk, 0)
        pltpu.sync_copy(acc_v, stage_sh.at[pl.ds(sid * _NPAD, _NPAD)])
        plsc.subcore_barrier()
        pltpu.sync_copy(stage_sh.at[pl.ds(sid * _RPT, _RPT)], red_v)
        for t in range(1, _NS):
            pltpu.sync_copy(
                stage_sh.at[pl.ds(t * _NPAD + sid * _RPT, _RPT)], tmp_v)
            for q in range(_RPT // 16):
                red_v[pl.ds(q * 16, 16)] = (
                    red_v[pl.ds(q * 16, 16)] + tmp_v[pl.ds(q * 16, 16)])
        pltpu.sync_copy(red_v, out_hbm.at[pl.ds(cid * _NPAD + sid * _RPT, _RPT)])

    return deg_kernel


def _make_scatter_kernel(F):
    @functools.partial(
        pl.kernel,
        out_type=jax.ShapeDtypeStruct((_NC, _NPAD, F), jnp.float32),
        mesh=_mesh,
        scratch_types=[
            pltpu.VMEM((_K,), jnp.int32),
            pltpu.VMEM((_K,), jnp.int32),
            pltpu.VMEM((_K, F), jnp.float32),
            pltpu.VMEM_SHARED((_NPAD, F), jnp.float32),
            pltpu.SemaphoreType.DMA,
        ],
    )
    def scat_kernel(vals_hbm, src_hbm, dst_hbm, zeros_hbm, out_hbm,
                    sidx_v, didx_v, rows_v, acc_sh, sem):
        cid = lax.axis_index("c")
        sid = lax.axis_index("s")
        wid = sid * _NC + cid
        pltpu.sync_copy(zeros_hbm, acc_sh.at[pl.ds(sid * _RPT, _RPT)])
        plsc.subcore_barrier()

        def chunk(ci, carry):
            base = wid * _EPW + ci * _K
            pltpu.sync_copy(src_hbm.at[pl.ds(base, _K)], sidx_v)
            pltpu.sync_copy(dst_hbm.at[pl.ds(base, _K)], didx_v)
            pltpu.async_copy(vals_hbm.at[sidx_v], rows_v, sem).wait()
            pltpu.sync_copy(rows_v, acc_sh.at[didx_v], add=True)
            return carry

        lax.fori_loop(0, _EPW // _K, chunk, 0)
        plsc.subcore_barrier()
        pltpu.sync_copy(
            acc_sh.at[pl.ds(sid * _RPT, _RPT)],
            out_hbm.at[cid, pl.ds(sid * _RPT, _RPT)],
        )

    return scat_kernel


_deg_kernel = _make_deg_kernel()
_scat128 = _make_scatter_kernel(128)


def _mm_xw_body(x_ref, w_ref, o_ref):
    o_ref[...] = jnp.dot(x_ref[...], w_ref[...], preferred_element_type=jnp.float32)


def _prep_body(degp_ref, xw_ref, xws_ref, dis_ref, inv_ref):
    deg = degp_ref[0] + degp_ref[1] + 1.0
    dis = lax.rsqrt(deg)
    inv = 1.0 / deg
    xws_ref[...] = xw_ref[...] * dis
    dis_ref[...] = jnp.broadcast_to(dis, (_ROWBLK, 128))
    inv_ref[...] = jnp.broadcast_to(inv, (_ROWBLK, 128))


def _mid_body(s1_ref, xw_ref, dis_ref, inv_ref, wm_ref, bc_ref, m_ref, ms_ref):
    hidden = (dis_ref[...] * (s1_ref[0] + s1_ref[1])
              + xw_ref[...] * inv_ref[...] + bc_ref[...])
    m = jnp.dot(hidden, wm_ref[...], preferred_element_type=jnp.float32)
    m_ref[...] = m
    ms = m * dis_ref[:, :64]
    ms_ref[...] = jnp.concatenate([ms, jnp.zeros_like(ms)], axis=1)


def _fin_body(s2_ref, m_ref, dis_ref, inv_ref, bm_ref, mu_ref, lv_ref):
    out2 = (dis_ref[:, :64] * (s2_ref[0, :, :64] + s2_ref[1, :, :64])
            + m_ref[...] * inv_ref[:, :64] + bm_ref[...])
    mu_ref[...] = out2[:, :32]
    lv_ref[...] = out2[:, 32:]


def _adj_body(mu_ref, mut_ref, o_ref):
    p = jnp.dot(mu_ref[...], mut_ref[...], preferred_element_type=jnp.float32)
    o_ref[...] = 0.5 * jnp.tanh(0.5 * p) + 0.5


def kernel(x, edge_index, W1, b1, W2, b2, Wmu, bmu, Wlv, blv):
    f32 = jnp.float32
    ei = jnp.asarray(edge_index, jnp.int32)
    src = ei[0]
    dst = ei[1]
    dpad = jnp.full((_EPAD - _E,), _NPAD - 1, jnp.int32)
    dst2 = jnp.concatenate([dst, dpad]).reshape(_EPAD // _KC, _KC)
    x_pad = jnp.pad(x.astype(f32), ((0, _NPAD - _N), (0, 0)))
    Wc = jnp.concatenate([W1, W2], axis=1).astype(f32)
    bc = jnp.concatenate([b1, b2]).reshape(1, 128).astype(f32)
    Wm = jnp.concatenate([Wmu, Wlv], axis=1).astype(f32)
    bm = jnp.concatenate([bmu, blv]).reshape(1, 64).astype(f32)

    zn = jnp.zeros((_NPAD,), f32)
    z128 = jnp.zeros((_RPT, 128), f32)

    degp = _deg_kernel(dst2, zn)
    degp = degp.reshape(_NC, _NPAD, 1)

    xw = pl.pallas_call(
        _mm_xw_body,
        grid=(_GRID,),
        in_specs=[
            pl.BlockSpec((_ROWBLK, 256), lambda i: (i, 0)),
            pl.BlockSpec((256, 128), lambda i: (0, 0)),
        ],
        out_specs=pl.BlockSpec((_ROWBLK, 128), lambda i: (i, 0)),
        out_shape=jax.ShapeDtypeStruct((_NPAD, 128), f32),
    )(x_pad, Wc)

    xws, dis_b, inv_b = pl.pallas_call(
        _prep_body,
        grid=(_GRID,),
        in_specs=[
            pl.BlockSpec((2, _ROWBLK, 1), lambda i: (0, i, 0)),
            pl.BlockSpec((_ROWBLK, 128), lambda i: (i, 0)),
        ],
        out_specs=[
            pl.BlockSpec((_ROWBLK, 128), lambda i: (i, 0)),
            pl.BlockSpec((_ROWBLK, 128), lambda i: (i, 0)),
            pl.BlockSpec((_ROWBLK, 128), lambda i: (i, 0)),
        ],
        out_shape=[
            jax.ShapeDtypeStruct((_NPAD, 128), f32),
            jax.ShapeDtypeStruct((_NPAD, 128), f32),
            jax.ShapeDtypeStruct((_NPAD, 128), f32),
        ],
    )(degp, xw)

    s1p = _scat128(xws, src, dst, z128)

    m, ms = pl.pallas_call(
        _mid_body,
        grid=(_GRID,),
        in_specs=[
            pl.BlockSpec((2, _ROWBLK, 128), lambda i: (0, i, 0)),
            pl.BlockSpec((_ROWBLK, 128), lambda i: (i, 0)),
            pl.BlockSpec((_ROWBLK, 128), lambda i: (i, 0)),
            pl.BlockSpec((_ROWBLK, 128), lambda i: (i, 0)),
            pl.BlockSpec((128, 64), lambda i: (0, 0)),
            pl.BlockSpec((1, 128), lambda i: (0, 0)),
        ],
        out_specs=[
            pl.BlockSpec((_ROWBLK, 64), lambda i: (i, 0)),
            pl.BlockSpec((_ROWBLK, 128), lambda i: (i, 0)),
        ],
        out_shape=[
            jax.ShapeDtypeStruct((_NPAD, 64), f32),
            jax.ShapeDtypeStruct((_NPAD, 128), f32),
        ],
    )(s1p, xw, dis_b, inv_b, Wm, bc)

    s2p = _scat128(ms, src, dst, z128)

    mu_full, lv_full = pl.pallas_call(
        _fin_body,
        grid=(_GRID,),
        in_specs=[
            pl.BlockSpec((2, _ROWBLK, 128), lambda i: (0, i, 0)),
            pl.BlockSpec((_ROWBLK, 64), lambda i: (i, 0)),
            pl.BlockSpec((_ROWBLK, 128), lambda i: (i, 0)),
            pl.BlockSpec((_ROWBLK, 128), lambda i: (i, 0)),
            pl.BlockSpec((1, 64), lambda i: (0, 0)),
        ],
        out_specs=[
            pl.BlockSpec((_ROWBLK, 32), lambda i: (i, 0)),
            pl.BlockSpec((_ROWBLK, 32), lambda i: (i, 0)),
        ],
        out_shape=[
            jax.ShapeDtypeStruct((_NPAD, 32), f32),
            jax.ShapeDtypeStruct((_NPAD, 32), f32),
        ],
    )(s2p, m, dis_b, inv_b, bm)

    mu = mu_full[:_N]
    logvar = lv_full[:_N]
    mut = mu.T

    adj_blk = 200
    adj = pl.pallas_call(
        _adj_body,
        grid=(_N // adj_blk,),
        in_specs=[
            pl.BlockSpec((adj_blk, 32), lambda i: (i, 0)),
            pl.BlockSpec((32, _N), lambda i: (0, 0)),
        ],
        out_specs=pl.BlockSpec((adj_blk, _N), lambda i: (i, 0)),
        out_shape=jax.ShapeDtypeStruct((_N, _N), f32),
    )(mu, mut)

    return (adj, mu, logvar)


# speedup vs baseline: 1.5761x; 1.1133x over previous
import functools

import jax
import jax.numpy as jnp
from jax import lax
from jax.experimental import pallas as pl
from jax.experimental.pallas import tpu as pltpu
from jax.experimental.pallas import tpu_sc as plsc

_N = 10000
_E = 320000
_NPAD = 10240
_NC = 2
_NS = 16
_KC = 128
_CPT = 80
_EPAD = _NC * _NS * _CPT * _KC
_K = 80
_EPW = _E // (_NC * _NS)
_RPT = _NPAD // _NS
_ROWBLK = 1024
_GRID = _NPAD // _ROWBLK

_mesh = plsc.VectorSubcoreMesh(core_axis_name="c", subcore_axis_name="s")


def _make_deg_kernel():
    @functools.partial(
        pl.kernel,
        out_type=jax.ShapeDtypeStruct((_NC * _NPAD,), jnp.float32),
        mesh=_mesh,
        scratch_types=[
            pltpu.VMEM((_CPT, _KC), jnp.int32),
            pltpu.VMEM((_NPAD,), jnp.float32),
            pltpu.VMEM((_RPT,), jnp.float32),
            pltpu.VMEM((_RPT,), jnp.float32),
            pltpu.VMEM_SHARED((_NS * _NPAD,), jnp.float32),
        ],
        compiler_params=pltpu.CompilerParams(needs_layout_passes=False),
    )
    def deg_kernel(dst_hbm, zeros_hbm, out_hbm, didx_v, acc_v, tmp_v, red_v, stage_sh):
        cid = lax.axis_index("c")
        sid = lax.axis_index("s")
        wid = sid * _NC + cid
        pltpu.sync_copy(dst_hbm.at[pl.ds(wid * _CPT, _CPT)], didx_v)
        pltpu.sync_copy(zeros_hbm, acc_v)
        ones16 = jnp.ones((16,), jnp.float32)

        def chunk(ci, carry):
            for g in range(_KC // 16):
                idx = didx_v[ci, pl.ds(g * 16, 16)]
                plsc.addupdate_scatter(acc_v, [idx], ones16)
            return carry

        lax.fori_loop(0, _CPT, chunk, 0)
        pltpu.sync_copy(acc_v, stage_sh.at[pl.ds(sid * _NPAD, _NPAD)])
        plsc.subcore_barrier()
        pltpu.sync_copy(stage_sh.at[pl.ds(sid * _RPT, _RPT)], red_v)
        for t in range(1, _NS):
            pltpu.sync_copy(
                stage_sh.at[pl.ds(t * _NPAD + sid * _RPT, _RPT)], tmp_v)
            for q in range(_RPT // 16):
                red_v[pl.ds(q * 16, 16)] = (
                    red_v[pl.ds(q * 16, 16)] + tmp_v[pl.ds(q * 16, 16)])
        pltpu.sync_copy(red_v, out_hbm.at[pl.ds(cid * _NPAD + sid * _RPT, _RPT)])

    return deg_kernel


def _make_scatter_kernel(F):
    @functools.partial(
        pl.kernel,
        out_type=jax.ShapeDtypeStruct((_NC, _NPAD, F), jnp.float32),
        mesh=_mesh,
        scratch_types=[
            pltpu.VMEM((_K,), jnp.int32),
            pltpu.VMEM((_K,), jnp.int32),
            pltpu.VMEM((_K,), jnp.int32),
            pltpu.VMEM((_K,), jnp.int32),
            pltpu.VMEM((_K, F), jnp.float32),
            pltpu.VMEM((_K, F), jnp.float32),
            pltpu.VMEM_SHARED((_NPAD, F), jnp.float32),
            pltpu.SemaphoreType.DMA,
            pltpu.SemaphoreType.DMA,
            pltpu.SemaphoreType.DMA,
            pltpu.SemaphoreType.DMA,
        ],
    )
    def scat_kernel(vals_hbm, src_hbm, dst_hbm, zeros_hbm, out_hbm,
                    sidx0, didx0, sidx1, didx1, rows0, rows1, acc_sh,
                    gs0, gs1, ss0, ss1):
        cid = lax.axis_index("c")
        sid = lax.axis_index("s")
        wid = sid * _NC + cid
        nchunks = _EPW // _K
        pltpu.sync_copy(zeros_hbm, acc_sh.at[pl.ds(sid * _RPT, _RPT)])

        def load_idx(ci, sv, dv):
            base = wid * _EPW + ci * _K
            pltpu.sync_copy(src_hbm.at[pl.ds(base, _K)], sv)
            pltpu.sync_copy(dst_hbm.at[pl.ds(base, _K)], dv)

        pltpu.sync_copy(zeros_hbm.at[pl.ds(0, _K)], rows0)
        pltpu.sync_copy(zeros_hbm.at[pl.ds(0, _K)], rows1)
        load_idx(0, sidx0, didx0)
        load_idx(0, sidx1, didx1)
        plsc.subcore_barrier()
        pltpu.async_copy(rows0, acc_sh.at[didx0], ss0, add=True)
        pltpu.async_copy(rows1, acc_sh.at[didx1], ss1, add=True)

        def step(ci, sv, dv, rv, gsem, ssem):
            pltpu.make_async_copy(rv, acc_sh.at[dv], ssem).wait()
            load_idx(ci, sv, dv)
            pltpu.async_copy(vals_hbm.at[sv], rv, gsem).wait()
            pltpu.async_copy(rv, acc_sh.at[dv], ssem, add=True)

        def pair(j, carry):
            step(2 * j, sidx0, didx0, rows0, gs0, ss0)
            step(2 * j + 1, sidx1, didx1, rows1, gs1, ss1)
            return carry

        lax.fori_loop(0, nchunks // 2, pair, 0)
        step(nchunks - 1, sidx0, didx0, rows0, gs0, ss0)
        pltpu.make_async_copy(rows0, acc_sh.at[didx0], ss0).wait()
        pltpu.make_async_copy(rows1, acc_sh.at[didx1], ss1).wait()
        plsc.subcore_barrier()
        pltpu.sync_copy(
            acc_sh.at[pl.ds(sid * _RPT, _RPT)],
            out_hbm.at[cid, pl.ds(sid * _RPT, _RPT)],
        )

    return scat_kernel


_deg_kernel = _make_deg_kernel()
_scat128 = _make_scatter_kernel(128)


def _mm_xw_body(x_ref, w_ref, o_ref):
    o_ref[...] = jnp.dot(x_ref[...], w_ref[...], preferred_element_type=jnp.float32)


def _prep_body(degp_ref, xw_ref, xws_ref, dis_ref, inv_ref):
    deg = degp_ref[0] + degp_ref[1] + 1.0
    dis = lax.rsqrt(deg)
    inv = 1.0 / deg
    xws_ref[...] = xw_ref[...] * dis
    dis_ref[...] = jnp.broadcast_to(dis, (_ROWBLK, 128))
    inv_ref[...] = jnp.broadcast_to(inv, (_ROWBLK, 128))


def _mid_body(s1_ref, xw_ref, dis_ref, inv_ref, wm_ref, bc_ref, m_ref, ms_ref):
    hidden = (dis_ref[...] * (s1_ref[0] + s1_ref[1])
              + xw_ref[...] * inv_ref[...] + bc_ref[...])
    m = jnp.dot(hidden, wm_ref[...], preferred_element_type=jnp.float32)
    m_ref[...] = m
    ms = m * dis_ref[:, :64]
    ms_ref[...] = jnp.concatenate([ms, jnp.zeros_like(ms)], axis=1)


def _fin_body(s2_ref, m_ref, dis_ref, inv_ref, bm_ref, mu_ref, lv_ref):
    out2 = (dis_ref[:, :64] * (s2_ref[0, :, :64] + s2_ref[1, :, :64])
            + m_ref[...] * inv_ref[:, :64] + bm_ref[...])
    mu_ref[...] = out2[:, :32]
    lv_ref[...] = out2[:, 32:]


def _adj_body(mu_ref, mut_ref, o_ref):
    p = jnp.dot(mu_ref[...], mut_ref[...], preferred_element_type=jnp.float32)
    o_ref[...] = 0.5 * jnp.tanh(0.5 * p) + 0.5


def kernel(x, edge_index, W1, b1, W2, b2, Wmu, bmu, Wlv, blv):
    f32 = jnp.float32
    ei = jnp.asarray(edge_index, jnp.int32)
    src = ei[0]
    dst = ei[1]
    dpad = jnp.full((_EPAD - _E,), _NPAD - 1, jnp.int32)
    dst2 = jnp.concatenate([dst, dpad]).reshape(_EPAD // _KC, _KC)
    x_pad = jnp.pad(x.astype(f32), ((0, _NPAD - _N), (0, 0)))
    Wc = jnp.concatenate([W1, W2], axis=1).astype(f32)
    bc = jnp.concatenate([b1, b2]).reshape(1, 128).astype(f32)
    Wm = jnp.concatenate([Wmu, Wlv], axis=1).astype(f32)
    bm = jnp.concatenate([bmu, blv]).reshape(1, 64).astype(f32)

    zn = jnp.zeros((_NPAD,), f32)
    z128 = jnp.zeros((_RPT, 128), f32)

    degp = _deg_kernel(dst2, zn)
    degp = degp.reshape(_NC, _NPAD, 1)

    xw = pl.pallas_call(
        _mm_xw_body,
        grid=(_GRID,),
        in_specs=[
            pl.BlockSpec((_ROWBLK, 256), lambda i: (i, 0)),
            pl.BlockSpec((256, 128), lambda i: (0, 0)),
        ],
        out_specs=pl.BlockSpec((_ROWBLK, 128), lambda i: (i, 0)),
        out_shape=jax.ShapeDtypeStruct((_NPAD, 128), f32),
    )(x_pad, Wc)

    xws, dis_b, inv_b = pl.pallas_call(
        _prep_body,
        grid=(_GRID,),
        in_specs=[
            pl.BlockSpec((2, _ROWBLK, 1), lambda i: (0, i, 0)),
            pl.BlockSpec((_ROWBLK, 128), lambda i: (i, 0)),
        ],
        out_specs=[
            pl.BlockSpec((_ROWBLK, 128), lambda i: (i, 0)),
            pl.BlockSpec((_ROWBLK, 128), lambda i: (i, 0)),
            pl.BlockSpec((_ROWBLK, 128), lambda i: (i, 0)),
        ],
        out_shape=[
            jax.ShapeDtypeStruct((_NPAD, 128), f32),
            jax.ShapeDtypeStruct((_NPAD, 128), f32),
            jax.ShapeDtypeStruct((_NPAD, 128), f32),
        ],
    )(degp, xw)

    s1p = _scat128(xws, src, dst, z128)

    m, ms = pl.pallas_call(
        _mid_body,
        grid=(_GRID,),
        in_specs=[
            pl.BlockSpec((2, _ROWBLK, 128), lambda i: (0, i, 0)),
            pl.BlockSpec((_ROWBLK, 128), lambda i: (i, 0)),
            pl.BlockSpec((_ROWBLK, 128), lambda i: (i, 0)),
            pl.BlockSpec((_ROWBLK, 128), lambda i: (i, 0)),
            pl.BlockSpec((128, 64), lambda i: (0, 0)),
            pl.BlockSpec((1, 128), lambda i: (0, 0)),
        ],
        out_specs=[
            pl.BlockSpec((_ROWBLK, 64), lambda i: (i, 0)),
            pl.BlockSpec((_ROWBLK, 128), lambda i: (i, 0)),
        ],
        out_shape=[
            jax.ShapeDtypeStruct((_NPAD, 64), f32),
            jax.ShapeDtypeStruct((_NPAD, 128), f32),
        ],
    )(s1p, xw, dis_b, inv_b, Wm, bc)

    s2p = _scat128(ms, src, dst, z128)

    mu_full, lv_full = pl.pallas_call(
        _fin_body,
        grid=(_GRID,),
        in_specs=[
            pl.BlockSpec((2, _ROWBLK, 128), lambda i: (0, i, 0)),
            pl.BlockSpec((_ROWBLK, 64), lambda i: (i, 0)),
            pl.BlockSpec((_ROWBLK, 128), lambda i: (i, 0)),
            pl.BlockSpec((_ROWBLK, 128), lambda i: (i, 0)),
            pl.BlockSpec((1, 64), lambda i: (0, 0)),
        ],
        out_specs=[
            pl.BlockSpec((_ROWBLK, 32), lambda i: (i, 0)),
            pl.BlockSpec((_ROWBLK, 32), lambda i: (i, 0)),
        ],
        out_shape=[
            jax.ShapeDtypeStruct((_NPAD, 32), f32),
            jax.ShapeDtypeStruct((_NPAD, 32), f32),
        ],
    )(s2p, m, dis_b, inv_b, bm)

    mu = mu_full[:_N]
    logvar = lv_full[:_N]
    mut = mu.T

    adj_blk = 200
    adj = pl.pallas_call(
        _adj_body,
        grid=(_N // adj_blk,),
        in_specs=[
            pl.BlockSpec((adj_blk, 32), lambda i: (i, 0)),
            pl.BlockSpec((32, _N), lambda i: (0, 0)),
        ],
        out_specs=pl.BlockSpec((adj_blk, _N), lambda i: (i, 0)),
        out_shape=jax.ShapeDtypeStruct((_N, _N), f32),
    )(mu, mut)

    return (adj, mu, logvar)
